# baseline (speedup 1.0000x reference)
import functools

import jax
import jax.numpy as jnp
from jax import lax
from jax.experimental import pallas as pl
from jax.experimental.pallas import tpu as pltpu
from jax.experimental.pallas import tpu_sc as plsc

N = 10000
E = 320000
D = 128

NTILES = 32
NSUB = 16
C = 128
KCH = 80
TOTCH = NTILES * KCH
EPAD = TOTCH * C
KS = 40
KF = 2 * KCH - KS
NP = 10240
RPT = NP // NSUB
BR = 256


def _mesh():
    return plsc.VectorSubcoreMesh(core_axis_name="c", subcore_axis_name="s",
                                  num_cores=2, num_subcores=NSUB)


def _deg_body(dsts_hbm, ones_hbm, zer_hbm, out_hbm, dst0, dst1, ones_v, acc,
              sem0, sem1):
    c = lax.axis_index("c")
    s = lax.axis_index("s")
    gid = c * NSUB + s
    lo = gid * KCH
    r0 = s * RPT
    pltpu.sync_copy(zer_hbm, acc.at[pl.ds(r0, RPT)])
    pltpu.sync_copy(ones_hbm, ones_v)
    plsc.subcore_barrier()
    pltpu.async_copy(dsts_hbm.at[lo], dst0, sem0)

    def body(jj, carry):
        j0 = lo + 2 * jj
        pltpu.async_copy(dsts_hbm.at[j0 + 1], dst1, sem1)
        pltpu.make_async_copy(dsts_hbm.at[j0], dst0, sem0).wait()
        pltpu.sync_copy(ones_v, acc.at[dst0], add=True)

        @pl.when(j0 + 2 < lo + KCH)
        def _():
            pltpu.async_copy(dsts_hbm.at[j0 + 2], dst0, sem0)

        pltpu.make_async_copy(dsts_hbm.at[j0 + 1], dst1, sem1).wait()
        pltpu.sync_copy(ones_v, acc.at[dst1], add=True)
        return carry

    lax.fori_loop(0, KCH // 2, body, 0)
    plsc.subcore_barrier()
    pltpu.sync_copy(acc.at[pl.ds(r0, RPT)], out_hbm.at[c, pl.ds(r0, RPT)])


def _deg_call(dsts, ones_rows, zer_rows):
    f = pl.kernel(
        _deg_body,
        out_type=jax.ShapeDtypeStruct((2, NP, D), jnp.float32),
        mesh=_mesh(),
        scratch_types=[
            pltpu.VMEM((C,), jnp.int32),
            pltpu.VMEM((C,), jnp.int32),
            pltpu.VMEM((C, D), jnp.float32),
            pltpu.VMEM_SHARED((NP, D), jnp.float32),
            pltpu.SemaphoreType.DMA,
            pltpu.SemaphoreType.DMA,
        ],
    )
    return f(dsts, ones_rows, zer_rows)


def _seg_body(g_hbm, srcs_hbm, dsts_hbm, zer_hbm, out_hbm,
              src0, src1, dst0, dst1, buf0, buf1, acc, sem0, sem1):
    c = lax.axis_index("c")
    s = lax.axis_index("s")
    r0 = s * RPT
    lo = jnp.where(c == 1, s * KS, NSUB * KS + s * KF)
    nch = jnp.where(c == 1, KS, KF)
    pltpu.sync_copy(zer_hbm, acc.at[pl.ds(r0, RPT)])
    plsc.subcore_barrier()
    pltpu.sync_copy(srcs_hbm.at[lo], src0)
    pltpu.sync_copy(dsts_hbm.at[lo], dst0)
    pltpu.async_copy(g_hbm.at[src0], buf0, sem0)

    def body(jj, carry):
        j0 = lo + 2 * jj
        pltpu.sync_copy(srcs_hbm.at[j0 + 1], src1)
        pltpu.sync_copy(dsts_hbm.at[j0 + 1], dst1)
        pltpu.async_copy(g_hbm.at[src1], buf1, sem1)
        pltpu.make_async_copy(g_hbm.at[src0], buf0, sem0).wait()
        pltpu.sync_copy(buf0, acc.at[dst0], add=True)

        @pl.when(j0 + 2 < lo + nch)
        def _():
            pltpu.sync_copy(srcs_hbm.at[j0 + 2], src0)
            pltpu.sync_copy(dsts_hbm.at[j0 + 2], dst0)
            pltpu.async_copy(g_hbm.at[src0], buf0, sem0)

        pltpu.make_async_copy(g_hbm.at[src1], buf1, sem1).wait()
        pltpu.sync_copy(buf1, acc.at[dst1], add=True)
        return carry

    lax.fori_loop(0, nch // 2, body, 0)
    plsc.subcore_barrier()
    pltpu.sync_copy(acc.at[pl.ds(r0, RPT)], out_hbm.at[c, pl.ds(r0, RPT)])


def _seg_call(g, srcs, dsts, zer_rows):
    f = pl.kernel(
        _seg_body,
        out_type=jax.ShapeDtypeStruct((2, NP, D), jnp.float32),
        mesh=_mesh(),
        scratch_types=[
            pltpu.VMEM((C,), jnp.int32),
            pltpu.VMEM((C,), jnp.int32),
            pltpu.VMEM((C,), jnp.int32),
            pltpu.VMEM((C,), jnp.int32),
            pltpu.VMEM((C, D), jnp.float32),
            pltpu.VMEM((C, D), jnp.float32),
            pltpu.VMEM_SHARED((NP, D), jnp.float32),
            pltpu.SemaphoreType.DMA,
            pltpu.SemaphoreType.DMA,
        ],
    )
    return f(g, srcs, dsts, zer_rows)


def _dinv_of(degp):
    deg = degp[0, :, 0:1] + degp[1, :, 0:1] + 1.0
    return lax.rsqrt(deg)


def _k1_body(x_ref, w_ref, degp_ref, o_ref):
    dinv = _dinv_of(degp_ref[...])
    o_ref[...] = dinv * jnp.dot(x_ref[...], w_ref[...],
                                preferred_element_type=jnp.float32)


def _k1_call(xp, W1, degp):
    return pl.pallas_call(
        _k1_body,
        grid=(NP // BR,),
        in_specs=[
            pl.BlockSpec((BR, D), lambda i: (i, 0)),
            pl.BlockSpec((D, D), lambda i: (0, 0)),
            pl.BlockSpec((2, BR, D), lambda i: (0, i, 0)),
        ],
        out_specs=pl.BlockSpec((BR, D), lambda i: (i, 0)),
        out_shape=jax.ShapeDtypeStruct((NP, D), jnp.float32),
    )(xp, W1, degp)


def _k2_body(part_ref, g1_ref, degp_ref, b_ref, w_ref, o_ref):
    i = pl.program_id(0)
    dinv = _dinv_of(degp_ref[...])
    p = part_ref[...]
    ssum = p[0] + p[1] + g1_ref[...]
    h = jnp.maximum(dinv * ssum + b_ref[...], 0.0)
    rows = i * BR + lax.broadcasted_iota(jnp.int32, (BR, 1), 0)
    h = jnp.where(rows < N, h, 0.0)
    o_ref[...] = dinv * jnp.dot(h, w_ref[...],
                                preferred_element_type=jnp.float32)


def _k2_call(part, g1, degp, b, W):
    return pl.pallas_call(
        _k2_body,
        grid=(NP // BR,),
        in_specs=[
            pl.BlockSpec((2, BR, D), lambda i: (0, i, 0)),
            pl.BlockSpec((BR, D), lambda i: (i, 0)),
            pl.BlockSpec((2, BR, D), lambda i: (0, i, 0)),
            pl.BlockSpec((1, D), lambda i: (0, 0)),
            pl.BlockSpec((D, D), lambda i: (0, 0)),
        ],
        out_specs=pl.BlockSpec((BR, D), lambda i: (i, 0)),
        out_shape=jax.ShapeDtypeStruct((NP, D), jnp.float32),
    )(part, g1, degp, b, W)


def _k3_body(part_ref, g2_ref, degp_ref, b_ref, lw_ref, lb_ref, o_ref):
    dinv = _dinv_of(degp_ref[...])
    p = part_ref[...]
    ssum = p[0] + p[1] + g2_ref[...]
    h = jnp.maximum(dinv * ssum + b_ref[...], 0.0)
    o_ref[...] = jnp.sum(h * lw_ref[...], axis=1, keepdims=True) + lb_ref[0, 0]


def _k3_call(part, g2, degp, b, lw_row, lb):
    return pl.pallas_call(
        _k3_body,
        grid=(NP // BR,),
        in_specs=[
            pl.BlockSpec((2, BR, D), lambda i: (0, i, 0)),
            pl.BlockSpec((BR, D), lambda i: (i, 0)),
            pl.BlockSpec((2, BR, D), lambda i: (0, i, 0)),
            pl.BlockSpec((1, D), lambda i: (0, 0)),
            pl.BlockSpec((1, D), lambda i: (0, 0)),
            pl.BlockSpec((1, 1), lambda i: (0, 0)),
        ],
        out_specs=pl.BlockSpec((BR, 1), lambda i: (i, 0)),
        out_shape=jax.ShapeDtypeStruct((NP, 1), jnp.float32),
    )(part, g2, degp, b, lw_row, lb)


def kernel(x, edge_index, W1, b1, W2, b2, lin_W, lin_b):
    src = edge_index[0]
    dst = edge_index[1]
    fill = jnp.full((EPAD - E,), N, dtype=jnp.int32)
    srcs = jnp.concatenate([src, fill]).reshape(TOTCH, C)
    dsts = jnp.concatenate([dst, fill]).reshape(TOTCH, C)
    xp = jnp.pad(x, ((0, NP - N), (0, 0)))
    ones_rows = jnp.ones((C, D), jnp.float32)
    zer128 = jnp.zeros((RPT, D), jnp.float32)

    degp = _deg_call(dsts, ones_rows, zer128)
    g1 = _k1_call(xp, W1, degp)
    part1 = _seg_call(g1, srcs, dsts, zer128)
    g2 = _k2_call(part1, g1, degp, b1.reshape(1, D), W2)
    part2 = _seg_call(g2, srcs, dsts, zer128)
    o = _k3_call(part2, g2, degp, b2.reshape(1, D),
                 lin_W.reshape(1, D), lin_b.reshape(1, 1))
    return o[:N, 0]


# speedup vs baseline: 2.6294x; 2.6294x over previous
import functools

import jax
import jax.numpy as jnp
from jax import lax
from jax.experimental import pallas as pl
from jax.experimental.pallas import tpu as pltpu
from jax.experimental.pallas import tpu_sc as plsc

N = 10000
E = 320000
D = 128

NTILES = 32
NSUB = 16
C = 128
KCH = 80
TOTCH = NTILES * KCH
EPAD = TOTCH * C
KS = 80
KF = 2 * KCH - KS
NP = 10240
RPT = NP // NSUB
BR = 256


def _mesh():
    return plsc.VectorSubcoreMesh(core_axis_name="c", subcore_axis_name="s",
                                  num_cores=2, num_subcores=NSUB)


def _deg_body(dsts_hbm, ones_hbm, zer_hbm, out_hbm, dst0, dst1, ones_v, acc,
              sem0, sem1):
    c = lax.axis_index("c")
    s = lax.axis_index("s")
    gid = c * NSUB + s
    lo = gid * KCH
    r0 = s * RPT
    pltpu.sync_copy(zer_hbm, acc.at[pl.ds(r0, RPT)])
    pltpu.sync_copy(ones_hbm, ones_v)
    plsc.subcore_barrier()
    pltpu.async_copy(dsts_hbm.at[lo], dst0, sem0)

    def body(jj, carry):
        j0 = lo + 2 * jj
        pltpu.async_copy(dsts_hbm.at[j0 + 1], dst1, sem1)
        pltpu.make_async_copy(dsts_hbm.at[j0], dst0, sem0).wait()
        pltpu.sync_copy(ones_v, acc.at[dst0], add=True)

        @pl.when(j0 + 2 < lo + KCH)
        def _():
            pltpu.async_copy(dsts_hbm.at[j0 + 2], dst0, sem0)

        pltpu.make_async_copy(dsts_hbm.at[j0 + 1], dst1, sem1).wait()
        pltpu.sync_copy(ones_v, acc.at[dst1], add=True)
        return carry

    lax.fori_loop(0, KCH // 2, body, 0)
    plsc.subcore_barrier()
    pltpu.sync_copy(acc.at[pl.ds(r0, RPT)], out_hbm.at[c, pl.ds(r0, RPT)])


def _deg_call(dsts, ones_rows, zer_rows):
    f = pl.kernel(
        _deg_body,
        out_type=jax.ShapeDtypeStruct((2, NP, D), jnp.float32),
        mesh=_mesh(),
        scratch_types=[
            pltpu.VMEM((C,), jnp.int32),
            pltpu.VMEM((C,), jnp.int32),
            pltpu.VMEM((C, D), jnp.float32),
            pltpu.VMEM_SHARED((NP, D), jnp.float32),
            pltpu.SemaphoreType.DMA,
            pltpu.SemaphoreType.DMA,
        ],
    )
    return f(dsts, ones_rows, zer_rows)


def _seg_body(g_hbm, srcs_hbm, dsts_hbm, zer_hbm, out_hbm,
              src0, src1, dst0, dst1, buf0, buf1, acc, sem0, sem1):
    c = lax.axis_index("c")
    s = lax.axis_index("s")
    r0 = s * RPT
    lo = jnp.where(c == 1, s * KS, NSUB * KS + s * KF)
    nch = jnp.where(c == 1, KS, KF)
    pltpu.sync_copy(zer_hbm, acc.at[pl.ds(r0, RPT)])
    plsc.subcore_barrier()
    pltpu.sync_copy(srcs_hbm.at[lo], src0)
    pltpu.sync_copy(dsts_hbm.at[lo], dst0)
    pltpu.async_copy(g_hbm.at[src0], buf0, sem0)

    def body(jj, carry):
        j0 = lo + 2 * jj
        pltpu.sync_copy(srcs_hbm.at[j0 + 1], src1)
        pltpu.sync_copy(dsts_hbm.at[j0 + 1], dst1)
        pltpu.async_copy(g_hbm.at[src1], buf1, sem1)
        pltpu.make_async_copy(g_hbm.at[src0], buf0, sem0).wait()
        pltpu.sync_copy(buf0, acc.at[dst0], add=True)

        @pl.when(j0 + 2 < lo + nch)
        def _():
            pltpu.sync_copy(srcs_hbm.at[j0 + 2], src0)
            pltpu.sync_copy(dsts_hbm.at[j0 + 2], dst0)
            pltpu.async_copy(g_hbm.at[src0], buf0, sem0)

        pltpu.make_async_copy(g_hbm.at[src1], buf1, sem1).wait()
        pltpu.sync_copy(buf1, acc.at[dst1], add=True)
        return carry

    lax.fori_loop(0, nch // 2, body, 0)
    plsc.subcore_barrier()
    pltpu.sync_copy(acc.at[pl.ds(r0, RPT)], out_hbm.at[c, pl.ds(r0, RPT)])


def _seg_call(g, srcs, dsts, zer_rows):
    f = pl.kernel(
        _seg_body,
        out_type=jax.ShapeDtypeStruct((2, NP, D), jnp.float32),
        mesh=_mesh(),
        scratch_types=[
            pltpu.VMEM((C,), jnp.int32),
            pltpu.VMEM((C,), jnp.int32),
            pltpu.VMEM((C,), jnp.int32),
            pltpu.VMEM((C,), jnp.int32),
            pltpu.VMEM((C, D), jnp.float32),
            pltpu.VMEM((C, D), jnp.float32),
            pltpu.VMEM_SHARED((NP, D), jnp.float32),
            pltpu.SemaphoreType.DMA,
            pltpu.SemaphoreType.DMA,
        ],
    )
    return f(g, srcs, dsts, zer_rows)


def _dinv_of(degp):
    deg = degp[0, :, 0:1] + degp[1, :, 0:1] + 1.0
    return lax.rsqrt(deg)


def _k1_body(x_ref, w_ref, degp_ref, o_ref):
    dinv = _dinv_of(degp_ref[...])
    o_ref[...] = dinv * jnp.dot(x_ref[...], w_ref[...],
                                preferred_element_type=jnp.float32)


def _k1_call(xp, W1, degp):
    return pl.pallas_call(
        _k1_body,
        grid=(NP // BR,),
        in_specs=[
            pl.BlockSpec((BR, D), lambda i: (i, 0)),
            pl.BlockSpec((D, D), lambda i: (0, 0)),
            pl.BlockSpec((2, BR, D), lambda i: (0, i, 0)),
        ],
        out_specs=pl.BlockSpec((BR, D), lambda i: (i, 0)),
        out_shape=jax.ShapeDtypeStruct((NP, D), jnp.float32),
    )(xp, W1, degp)


def _k2_body(part_ref, g1_ref, degp_ref, b_ref, w_ref, o_ref):
    i = pl.program_id(0)
    dinv = _dinv_of(degp_ref[...])
    p = part_ref[...]
    ssum = p[0] + p[1] + g1_ref[...]
    h = jnp.maximum(dinv * ssum + b_ref[...], 0.0)
    rows = i * BR + lax.broadcasted_iota(jnp.int32, (BR, 1), 0)
    h = jnp.where(rows < N, h, 0.0)
    o_ref[...] = dinv * jnp.dot(h, w_ref[...],
                                preferred_element_type=jnp.float32)


def _k2_call(part, g1, degp, b, W):
    return pl.pallas_call(
        _k2_body,
        grid=(NP // BR,),
        in_specs=[
            pl.BlockSpec((2, BR, D), lambda i: (0, i, 0)),
            pl.BlockSpec((BR, D), lambda i: (i, 0)),
            pl.BlockSpec((2, BR, D), lambda i: (0, i, 0)),
            pl.BlockSpec((1, D), lambda i: (0, 0)),
            pl.BlockSpec((D, D), lambda i: (0, 0)),
        ],
        out_specs=pl.BlockSpec((BR, D), lambda i: (i, 0)),
        out_shape=jax.ShapeDtypeStruct((NP, D), jnp.float32),
    )(part, g1, degp, b, W)


def _k3_body(part_ref, g2_ref, degp_ref, b_ref, lw_ref, lb_ref, o_ref):
    dinv = _dinv_of(degp_ref[...])
    p = part_ref[...]
    ssum = p[0] + p[1] + g2_ref[...]
    h = jnp.maximum(dinv * ssum + b_ref[...], 0.0)
    o_ref[...] = jnp.sum(h * lw_ref[...], axis=1, keepdims=True) + lb_ref[0, 0]


def _k3_call(part, g2, degp, b, lw_row, lb):
    return pl.pallas_call(
        _k3_body,
        grid=(NP // BR,),
        in_specs=[
            pl.BlockSpec((2, BR, D), lambda i: (0, i, 0)),
            pl.BlockSpec((BR, D), lambda i: (i, 0)),
            pl.BlockSpec((2, BR, D), lambda i: (0, i, 0)),
            pl.BlockSpec((1, D), lambda i: (0, 0)),
            pl.BlockSpec((1, D), lambda i: (0, 0)),
            pl.BlockSpec((1, 1), lambda i: (0, 0)),
        ],
        out_specs=pl.BlockSpec((BR, 1), lambda i: (i, 0)),
        out_shape=jax.ShapeDtypeStruct((NP, 1), jnp.float32),
    )(part, g2, degp, b, lw_row, lb)


def kernel(x, edge_index, W1, b1, W2, b2, lin_W, lin_b):
    src = edge_index[0]
    dst = edge_index[1]
    fill = N + (jnp.arange(EPAD - E, dtype=jnp.int32) % (NP - N))
    srcs = jnp.concatenate([src, fill]).reshape(TOTCH, C)
    dsts = jnp.concatenate([dst, fill]).reshape(TOTCH, C)
    xp = jnp.pad(x, ((0, NP - N), (0, 0)))
    ones_rows = jnp.ones((C, D), jnp.float32)
    zer128 = jnp.zeros((RPT, D), jnp.float32)

    degp = _deg_call(dsts, ones_rows, zer128)
    g1 = _k1_call(xp, W1, degp)
    part1 = _seg_call(g1, srcs, dsts, zer128)
    g2 = _k2_call(part1, g1, degp, b1.reshape(1, D), W2)
    part2 = _seg_call(g2, srcs, dsts, zer128)
    o = _k3_call(part2, g2, degp, b2.reshape(1, D),
                 lin_W.reshape(1, D), lin_b.reshape(1, 1))
    return o[:N, 0]


# speedup vs baseline: 2.8830x; 1.0965x over previous
import jax
import jax.numpy as jnp
from jax import lax
from jax.experimental import pallas as pl
from jax.experimental.pallas import tpu as pltpu
from jax.experimental.pallas import tpu_sc as plsc

N = 10000
E = 320000
D = 128

NTILES = 32
NSUB = 16
C = 128
TOTCH = E // C
NP = 10240
RPT = NP // NSUB
BR = 512


def _mesh():
    return plsc.VectorSubcoreMesh(core_axis_name="c", subcore_axis_name="s",
                                  num_cores=2, num_subcores=NSUB)


def _chunk_range(gid):
    lo = gid * 78 + jnp.minimum(gid, 2) * 2
    nch = jnp.where(gid < 2, 80, 78)
    return lo, nch


def _deg_body(edges_hbm, ones_hbm, zer_hbm, out_hbm, dst0, dst1, ones_v, acc,
              sem0, sem1):
    c = lax.axis_index("c")
    s = lax.axis_index("s")
    gid = c * NSUB + s
    lo, nch = _chunk_range(gid)
    r0 = s * RPT
    pltpu.sync_copy(zer_hbm, acc.at[pl.ds(r0, RPT)])
    pltpu.sync_copy(ones_hbm, ones_v)
    plsc.subcore_barrier()
    pltpu.async_copy(edges_hbm.at[1, lo], dst0, sem0)

    def body(jj, carry):
        j0 = lo + 2 * jj
        pltpu.async_copy(edges_hbm.at[1, j0 + 1], dst1, sem1)
        pltpu.make_async_copy(edges_hbm.at[1, j0], dst0, sem0).wait()
        pltpu.sync_copy(ones_v, acc.at[dst0], add=True)

        @pl.when(j0 + 2 < lo + nch)
        def _():
            pltpu.async_copy(edges_hbm.at[1, j0 + 2], dst0, sem0)

        pltpu.make_async_copy(edges_hbm.at[1, j0 + 1], dst1, sem1).wait()
        pltpu.sync_copy(ones_v, acc.at[dst1], add=True)
        return carry

    lax.fori_loop(0, nch // 2, body, 0)
    plsc.subcore_barrier()
    pltpu.sync_copy(acc.at[pl.ds(r0, RPT)], out_hbm.at[c, pl.ds(r0, RPT)])


def _deg_call(edges, ones_rows, zer_rows):
    f = pl.kernel(
        _deg_body,
        out_type=jax.ShapeDtypeStruct((2, NP, D), jnp.float32),
        mesh=_mesh(),
        scratch_types=[
            pltpu.VMEM((C,), jnp.int32),
            pltpu.VMEM((C,), jnp.int32),
            pltpu.VMEM((C, D), jnp.float32),
            pltpu.VMEM_SHARED((NP, D), jnp.float32),
            pltpu.SemaphoreType.DMA,
            pltpu.SemaphoreType.DMA,
        ],
    )
    return f(edges, ones_rows, zer_rows)


def _seg_body(g_hbm, edges_hbm, zer_hbm, out_hbm,
              src0, src1, dst0, dst1, buf0, buf1, acc, sem0, sem1):
    c = lax.axis_index("c")
    s = lax.axis_index("s")
    gid = c * NSUB + s
    lo, nch = _chunk_range(gid)
    r0 = s * RPT
    pltpu.sync_copy(zer_hbm, acc.at[pl.ds(r0, RPT)])
    plsc.subcore_barrier()
    pltpu.sync_copy(edges_hbm.at[0, lo], src0)
    pltpu.sync_copy(edges_hbm.at[1, lo], dst0)
    pltpu.async_copy(g_hbm.at[src0], buf0, sem0)

    def body(jj, carry):
        j0 = lo + 2 * jj
        pltpu.sync_copy(edges_hbm.at[0, j0 + 1], src1)
        pltpu.sync_copy(edges_hbm.at[1, j0 + 1], dst1)
        pltpu.async_copy(g_hbm.at[src1], buf1, sem1)
        pltpu.make_async_copy(g_hbm.at[src0], buf0, sem0).wait()
        pltpu.sync_copy(buf0, acc.at[dst0], add=True)

        @pl.when(j0 + 2 < lo + nch)
        def _():
            pltpu.sync_copy(edges_hbm.at[0, j0 + 2], src0)
            pltpu.sync_copy(edges_hbm.at[1, j0 + 2], dst0)
            pltpu.async_copy(g_hbm.at[src0], buf0, sem0)

        pltpu.make_async_copy(g_hbm.at[src1], buf1, sem1).wait()
        pltpu.sync_copy(buf1, acc.at[dst1], add=True)
        return carry

    lax.fori_loop(0, nch // 2, body, 0)
    plsc.subcore_barrier()
    pltpu.sync_copy(acc.at[pl.ds(r0, RPT)], out_hbm.at[c, pl.ds(r0, RPT)])


def _seg_call(g, edges, zer_rows):
    f = pl.kernel(
        _seg_body,
        out_type=jax.ShapeDtypeStruct((2, NP, D), jnp.float32),
        mesh=_mesh(),
        scratch_types=[
            pltpu.VMEM((C,), jnp.int32),
            pltpu.VMEM((C,), jnp.int32),
            pltpu.VMEM((C,), jnp.int32),
            pltpu.VMEM((C,), jnp.int32),
            pltpu.VMEM((C, D), jnp.float32),
            pltpu.VMEM((C, D), jnp.float32),
            pltpu.VMEM_SHARED((NP, D), jnp.float32),
            pltpu.SemaphoreType.DMA,
            pltpu.SemaphoreType.DMA,
        ],
    )
    return f(g, edges, zer_rows)


def _dinv_of(degp):
    deg = degp[0, :, 0:1] + degp[1, :, 0:1] + 1.0
    return lax.rsqrt(deg)


def _k1a_body(x_ref, w_ref, o_ref):
    o_ref[...] = jnp.dot(x_ref[...], w_ref[...],
                         preferred_element_type=jnp.float32)


def _k1a_call(xp, W1):
    return pl.pallas_call(
        _k1a_body,
        grid=(NP // BR,),
        in_specs=[
            pl.BlockSpec((BR, D), lambda i: (i, 0)),
            pl.BlockSpec((D, D), lambda i: (0, 0)),
        ],
        out_specs=pl.BlockSpec((BR, D), lambda i: (i, 0)),
        out_shape=jax.ShapeDtypeStruct((NP, D), jnp.float32),
    )(xp, W1)


def _k1b_body(xw_ref, degp_ref, o_ref):
    dinv = _dinv_of(degp_ref[...])
    o_ref[...] = dinv * xw_ref[...]


def _k1b_call(xw, degp):
    return pl.pallas_call(
        _k1b_body,
        grid=(NP // BR,),
        in_specs=[
            pl.BlockSpec((BR, D), lambda i: (i, 0)),
            pl.BlockSpec((2, BR, D), lambda i: (0, i, 0)),
        ],
        out_specs=pl.BlockSpec((BR, D), lambda i: (i, 0)),
        out_shape=jax.ShapeDtypeStruct((NP, D), jnp.float32),
    )(xw, degp)


def _k2_body(part_ref, g1_ref, degp_ref, b_ref, w_ref, o_ref):
    i = pl.program_id(0)
    dinv = _dinv_of(degp_ref[...])
    p = part_ref[...]
    ssum = p[0] + p[1] + g1_ref[...]
    h = jnp.maximum(dinv * ssum + b_ref[...], 0.0)
    rows = i * BR + lax.broadcasted_iota(jnp.int32, (BR, 1), 0)
    h = jnp.where(rows < N, h, 0.0)
    o_ref[...] = dinv * jnp.dot(h, w_ref[...],
                                preferred_element_type=jnp.float32)


def _k2_call(part, g1, degp, b, W):
    return pl.pallas_call(
        _k2_body,
        grid=(NP // BR,),
        in_specs=[
            pl.BlockSpec((2, BR, D), lambda i: (0, i, 0)),
            pl.BlockSpec((BR, D), lambda i: (i, 0)),
            pl.BlockSpec((2, BR, D), lambda i: (0, i, 0)),
            pl.BlockSpec((1, D), lambda i: (0, 0)),
            pl.BlockSpec((D, D), lambda i: (0, 0)),
        ],
        out_specs=pl.BlockSpec((BR, D), lambda i: (i, 0)),
        out_shape=jax.ShapeDtypeStruct((NP, D), jnp.float32),
    )(part, g1, degp, b, W)


def _k3_body(part_ref, g2_ref, degp_ref, b_ref, lw_ref, lb_ref, o_ref):
    dinv = _dinv_of(degp_ref[...])
    p = part_ref[...]
    ssum = p[0] + p[1] + g2_ref[...]
    h = jnp.maximum(dinv * ssum + b_ref[...], 0.0)
    o_ref[...] = jnp.sum(h * lw_ref[...], axis=1, keepdims=True) + lb_ref[0, 0]


def _k3_call(part, g2, degp, b, lw_row, lb):
    return pl.pallas_call(
        _k3_body,
        grid=(NP // BR,),
        in_specs=[
            pl.BlockSpec((2, BR, D), lambda i: (0, i, 0)),
            pl.BlockSpec((BR, D), lambda i: (i, 0)),
            pl.BlockSpec((2, BR, D), lambda i: (0, i, 0)),
            pl.BlockSpec((1, D), lambda i: (0, 0)),
            pl.BlockSpec((1, D), lambda i: (0, 0)),
            pl.BlockSpec((1, 1), lambda i: (0, 0)),
        ],
        out_specs=pl.BlockSpec((BR, 1), lambda i: (i, 0)),
        out_shape=jax.ShapeDtypeStruct((NP, 1), jnp.float32),
    )(part, g2, degp, b, lw_row, lb)


def kernel(x, edge_index, W1, b1, W2, b2, lin_W, lin_b):
    edges = edge_index.reshape(2, TOTCH, C)
    xp = jnp.pad(x, ((0, NP - N), (0, 0)))
    ones_rows = jnp.ones((C, D), jnp.float32)
    zer128 = jnp.zeros((RPT, D), jnp.float32)

    degp = _deg_call(edges, ones_rows, zer128)
    xw1 = _k1a_call(xp, W1)
    g1 = _k1b_call(xw1, degp)
    part1 = _seg_call(g1, edges, zer128)
    g2 = _k2_call(part1, g1, degp, b1.reshape(1, D), W2)
    part2 = _seg_call(g2, edges, zer128)
    o = _k3_call(part2, g2, degp, b2.reshape(1, D),
                 lin_W.reshape(1, D), lin_b.reshape(1, 1))
    return o[:N, 0]
